# baseline (speedup 1.0000x reference)
import functools

import jax
import jax.numpy as jnp
from jax import lax
from jax.experimental import pallas as pl
from jax.experimental.pallas import tpu as pltpu
from jax.experimental.pallas import tpu_sc as plsc

DIM = 64
CH = 128
Q = 4
BLK4 = 1024


def _sc_gather(idx3, table):
    info = plsc.get_sparse_core_info()
    nw = info.num_cores * info.num_subcores
    n_ch = idx3.shape[1]
    b_per_w = n_ch * CH
    n = nw * b_per_w
    slab = next(s for s in range(1024, 0, -CH) if b_per_w % s == 0)
    n_slab = b_per_w // slab
    k = slab // CH

    mesh = plsc.VectorSubcoreMesh(core_axis_name="c", subcore_axis_name="s")

    @functools.partial(
        pl.kernel,
        mesh=mesh,
        out_type=jax.ShapeDtypeStruct((n, DIM), jnp.bfloat16),
        scratch_types=[
            pltpu.VMEM((n_ch, CH), jnp.int32),
            pltpu.VMEM((slab, DIM), jnp.bfloat16),
            pltpu.SemaphoreType.DMA,
        ],
        compiler_params=pltpu.CompilerParams(use_tc_tiling_on_sc=False),
    )
    def kern(idx_hbm, table_hbm, out_hbm, idx_v, rows_v, sem):
        wid = lax.axis_index("s") * info.num_cores + lax.axis_index("c")
        base = wid * b_per_w
        pltpu.sync_copy(idx_hbm.at[wid], idx_v)

        def body(s, carry):
            cops = []
            for j in range(k):
                cop = pltpu.async_copy(
                    table_hbm.at[idx_v.at[s * k + j]],
                    rows_v.at[pl.ds(j * CH, CH)],
                    sem,
                )
                cops.append(cop)
            for cop in cops:
                cop.wait()
            pltpu.sync_copy(rows_v, out_hbm.at[pl.ds(base + s * slab, slab)])
            return carry

        lax.fori_loop(0, n_slab, body, 0)

    return kern(idx3, table)


def _pack4(w):
    z = jnp.zeros((DIM, DIM), jnp.float32)
    rows = []
    for i in range(4):
        blocks = [w if j == i else z for j in range(4)]
        rows.append(jnp.concatenate(blocks, axis=1))
    return jnp.concatenate(rows, axis=0).astype(jnp.bfloat16)


def _tc_forward(emb2, weights, out_buf, blk0, n_total):
    n2 = emb2.shape[0]
    grid = n2 // (2 * BLK4)

    def body(xa_ref, xb_ref, wp_ref, wg0_ref, bg0_ref, wt0_ref, bt0_ref,
             wg1_ref, bg1_ref, wt1_ref, bt1_ref, *rest):
        o_ref = rest[-1]
        x4 = jnp.concatenate([xa_ref[...], xb_ref[...]], axis=1)
        h = jnp.dot(x4, wp_ref[...], preferred_element_type=jnp.float32)
        for wg, bg, wt, bt in ((wg0_ref, bg0_ref, wt0_ref, bt0_ref),
                               (wg1_ref, bg1_ref, wt1_ref, bt1_ref)):
            hb = h.astype(jnp.bfloat16)
            z = jnp.dot(hb, wg[...], preferred_element_type=jnp.float32) + bg[...]
            g = 1.0 / (1.0 + jnp.exp(-z))
            t = jnp.maximum(
                jnp.dot(hb, wt[...], preferred_element_type=jnp.float32)
                + bt[...], 0.0)
            h = g * t + (1.0 - g) * h
        o_ref[...] = jnp.concatenate(
            [h[:, c * DIM:(c + 1) * DIM] for c in range(4)], axis=0)

    w_spec = pl.BlockSpec((4 * DIM, 4 * DIM), lambda i: (0, 0))
    b_spec = pl.BlockSpec((1, 4 * DIM), lambda i: (0, 0))
    in_specs = [
        pl.BlockSpec((BLK4, 2 * DIM), lambda i: (2 * i, 0)),
        pl.BlockSpec((BLK4, 2 * DIM), lambda i: (2 * i + 1, 0)),
        w_spec, w_spec, b_spec, w_spec, b_spec,
        w_spec, b_spec, w_spec, b_spec,
    ]
    operands = [emb2, emb2, *weights]
    io_alias = {}
    if out_buf is not None:
        in_specs.append(pl.BlockSpec(memory_space=pl.ANY))
        operands.append(out_buf)
        io_alias = {len(operands) - 1: 0}

    return pl.pallas_call(
        body,
        grid=(grid,),
        in_specs=in_specs,
        out_specs=pl.BlockSpec((4 * BLK4, DIM), lambda i: (blk0 + i, 0)),
        out_shape=jax.ShapeDtypeStruct((n_total, DIM), jnp.float32),
        input_output_aliases=io_alias,
    )(*operands)


def kernel(x, word_vectors, W_proj, Wt0, bt0, Wg0, bg0, Wt1, bt1, Wg1, bg1):
    b, l = x.shape
    n = b * l
    per = n // Q
    nw = 32
    idx_flat = x.reshape(-1).astype(jnp.int32)
    idx_flat = idx_flat.reshape(-1, 2, 2, BLK4).transpose(0, 1, 3, 2).reshape(-1)
    idx4 = idx_flat.reshape(Q, nw, per // (nw * CH), CH)

    def d4(v):
        return jnp.concatenate([v, v, v, v]).reshape(1, 4 * DIM)

    weights = (
        _pack4(W_proj.T),
        _pack4(Wg0.T), d4(bg0), _pack4(Wt0.T), d4(bt0),
        _pack4(Wg1.T), d4(bg1), _pack4(Wt1.T), d4(bt1),
    )

    table_bf = word_vectors.astype(jnp.bfloat16)
    embs = [_sc_gather(idx4[q], table_bf) for q in range(Q)]
    out = None
    for q in range(Q):
        emb2 = embs[q].reshape(per // 2, 2 * DIM)
        out = _tc_forward(emb2, weights, out, q * (per // (4 * BLK4)), n)
    return out.reshape(b, l, DIM)


# speedup vs baseline: 1.7264x; 1.7264x over previous
import functools

import jax
import jax.numpy as jnp
from jax import lax
from jax.experimental import pallas as pl
from jax.experimental.pallas import tpu as pltpu
from jax.experimental.pallas import tpu_sc as plsc

DIM = 64
CH = 128
Q = 4
BLK4 = 1600


def _sc_gather(idx2, table):
    info = plsc.get_sparse_core_info()
    nw = info.num_cores * info.num_subcores
    b_per_w = idx2.shape[1]
    n_ch = b_per_w // CH
    n = nw * b_per_w
    half = b_per_w // 2
    b4 = b_per_w // 4
    n_grp = b_per_w // 16
    slab = next(s for s in range(1024, 0, -CH) if b_per_w % s == 0)
    n_slab = b_per_w // slab
    k = slab // CH

    mesh = plsc.VectorSubcoreMesh(core_axis_name="c", subcore_axis_name="s")

    @functools.partial(
        pl.kernel,
        mesh=mesh,
        out_type=jax.ShapeDtypeStruct((n, DIM), jnp.float32),
        scratch_types=[
            pltpu.VMEM((b_per_w,), jnp.int32),
            pltpu.VMEM((n_ch, CH), jnp.int32),
            pltpu.VMEM((slab, DIM), jnp.float32),
            pltpu.SemaphoreType.DMA,
        ],
        compiler_params=pltpu.CompilerParams(use_tc_tiling_on_sc=False,
                                             needs_layout_passes=False),
    )
    def kern(idx_hbm, table_hbm, out_hbm, idx_v, idx_p, rows_v, sem):
        wid = lax.axis_index("s") * info.num_cores + lax.axis_index("c")
        base = wid * b_per_w
        pltpu.sync_copy(idx_hbm.at[wid], idx_v)

        lane = lax.iota(jnp.int32, 16)
        patt = (lane >> 1) + (lane & 1) * b4

        def permute(g, carry):
            u0 = g * 16
            a = u0 // half
            j0 = (u0 % half) // 2
            v = plsc.load_gather(idx_v, [patt + (a * half + j0)])
            idx_p[g // 8, pl.ds((g % 8) * 16, 16)] = v
            return carry

        lax.fori_loop(0, n_grp, permute, 0)

        def body(s, carry):
            cops = []
            for j in range(k):
                cop = pltpu.async_copy(
                    table_hbm.at[idx_p.at[s * k + j]],
                    rows_v.at[pl.ds(j * CH, CH)],
                    sem,
                )
                cops.append(cop)
            for cop in cops:
                cop.wait()
            pltpu.sync_copy(rows_v, out_hbm.at[pl.ds(base + s * slab, slab)])
            return carry

        lax.fori_loop(0, n_slab, body, 0)

    return kern(idx2, table)


def _pack4(w):
    z = jnp.zeros((DIM, DIM), jnp.float32)
    rows = []
    for i in range(4):
        blocks = [w if j == i else z for j in range(4)]
        rows.append(jnp.concatenate(blocks, axis=1))
    return jnp.concatenate(rows, axis=0).astype(jnp.bfloat16)


def _tc_forward(emb2, weights, out_buf, blk0, n_total):
    n2 = emb2.shape[0]
    grid = n2 // (2 * BLK4)

    def body(xa_ref, xb_ref, wp_ref, wg0_ref, bg0_ref, wt0_ref, bt0_ref,
             wg1_ref, bg1_ref, wt1_ref, bt1_ref, *rest):
        o_ref = rest[-1]
        x4 = jnp.concatenate([xa_ref[...], xb_ref[...]], axis=1)
        h = jnp.dot(x4.astype(jnp.bfloat16), wp_ref[...],
                    preferred_element_type=jnp.float32)
        for wg, bg, wt, bt in ((wg0_ref, bg0_ref, wt0_ref, bt0_ref),
                               (wg1_ref, bg1_ref, wt1_ref, bt1_ref)):
            hb = h.astype(jnp.bfloat16)
            z = jnp.dot(hb, wg[...], preferred_element_type=jnp.float32) + bg[...]
            g = 1.0 / (1.0 + jnp.exp(-z))
            t = jnp.maximum(
                jnp.dot(hb, wt[...], preferred_element_type=jnp.float32)
                + bt[...], 0.0)
            h = g * t + (1.0 - g) * h
        o_ref[...] = jnp.concatenate(
            [h[:, c * DIM:(c + 1) * DIM] for c in range(4)], axis=0)

    w_spec = pl.BlockSpec((4 * DIM, 4 * DIM), lambda i: (0, 0))
    b_spec = pl.BlockSpec((1, 4 * DIM), lambda i: (0, 0))
    in_specs = [
        pl.BlockSpec((BLK4, 2 * DIM), lambda i: (2 * i, 0)),
        pl.BlockSpec((BLK4, 2 * DIM), lambda i: (2 * i + 1, 0)),
        w_spec, w_spec, b_spec, w_spec, b_spec,
        w_spec, b_spec, w_spec, b_spec,
    ]
    operands = [emb2, emb2, *weights]
    io_alias = {}
    if out_buf is not None:
        in_specs.append(pl.BlockSpec(memory_space=pl.ANY))
        operands.append(out_buf)
        io_alias = {len(operands) - 1: 0}

    return pl.pallas_call(
        body,
        grid=(grid,),
        in_specs=in_specs,
        out_specs=pl.BlockSpec((4 * BLK4, DIM), lambda i: (blk0 + i, 0)),
        out_shape=jax.ShapeDtypeStruct((n_total, DIM), jnp.float32),
        input_output_aliases=io_alias,
    )(*operands)


def kernel(x, word_vectors, W_proj, Wt0, bt0, Wg0, bg0, Wt1, bt1, Wg1, bg1):
    b, l = x.shape
    n = b * l
    per = n // Q
    nw = 32
    idx_flat = x.reshape(-1).astype(jnp.int32)
    idx4 = idx_flat.reshape(Q, nw, per // nw)

    def d4(v):
        return jnp.concatenate([v, v, v, v]).reshape(1, 4 * DIM)

    weights = (
        _pack4(W_proj.T),
        _pack4(Wg0.T), d4(bg0), _pack4(Wt0.T), d4(bt0),
        _pack4(Wg1.T), d4(bg1), _pack4(Wt1.T), d4(bt1),
    )

    embs = [_sc_gather(idx4[q], word_vectors) for q in range(Q)]
    out = None
    for q in range(Q):
        emb2 = embs[q].reshape(per // 2, 2 * DIM)
        out = _tc_forward(emb2, weights, out, q * (per // (4 * BLK4)), n)
    return out.reshape(b, l, DIM)


# speedup vs baseline: 1.7275x; 1.0007x over previous
import functools

import jax
import jax.numpy as jnp
from jax import lax
from jax.experimental import pallas as pl
from jax.experimental.pallas import tpu as pltpu
from jax.experimental.pallas import tpu_sc as plsc

DIM = 64
CH = 128
Q = 4
BLK4 = 1600


def _sc_gather(idx2, table):
    info = plsc.get_sparse_core_info()
    nw = info.num_cores * info.num_subcores
    n_row = idx2.shape[0]
    b_per_w = n_row * CH // nw
    rows_w = b_per_w // CH
    n_ch = b_per_w // CH
    n = nw * b_per_w
    half = b_per_w // 2
    b4 = b_per_w // 4
    n_grp = b_per_w // 16
    slab = next(s for s in range(1024, 0, -CH) if b_per_w % s == 0)
    n_slab = b_per_w // slab
    k = slab // CH

    mesh = plsc.VectorSubcoreMesh(core_axis_name="c", subcore_axis_name="s")

    @functools.partial(
        pl.kernel,
        mesh=mesh,
        out_type=jax.ShapeDtypeStruct((n, DIM), jnp.float32),
        scratch_types=[
            pltpu.VMEM((rows_w, CH), jnp.int32),
            pltpu.VMEM((n_ch, CH), jnp.int32),
            pltpu.VMEM((slab, DIM), jnp.float32),
            pltpu.SemaphoreType.DMA,
        ],
        compiler_params=pltpu.CompilerParams(use_tc_tiling_on_sc=False,
                                             needs_layout_passes=False),
    )
    def kern(idx_hbm, table_hbm, out_hbm, idx_v, idx_p, rows_v, sem):
        wid = lax.axis_index("s") * info.num_cores + lax.axis_index("c")
        base = wid * b_per_w
        pltpu.sync_copy(idx_hbm.at[pl.ds(wid * rows_w, rows_w)], idx_v)

        lane = lax.iota(jnp.int32, 16)
        patt = (lane >> 1) + (lane & 1) * b4

        def permute(g, carry):
            u0 = g * 16
            a = u0 // half
            j0 = (u0 % half) // 2
            p = patt + (a * half + j0)
            v = plsc.load_gather(idx_v, [p >> 7, p & (CH - 1)])
            idx_p[g // 8, pl.ds((g % 8) * 16, 16)] = v
            return carry

        lax.fori_loop(0, n_grp, permute, 0)

        def body(s, carry):
            cops = []
            for j in range(k):
                cop = pltpu.async_copy(
                    table_hbm.at[idx_p.at[s * k + j]],
                    rows_v.at[pl.ds(j * CH, CH)],
                    sem,
                )
                cops.append(cop)
            for cop in cops:
                cop.wait()
            pltpu.sync_copy(rows_v, out_hbm.at[pl.ds(base + s * slab, slab)])
            return carry

        lax.fori_loop(0, n_slab, body, 0)

    return kern(idx2, table)


def _pack4(w):
    z = jnp.zeros((DIM, DIM), jnp.float32)
    rows = []
    for i in range(4):
        blocks = [w if j == i else z for j in range(4)]
        rows.append(jnp.concatenate(blocks, axis=1))
    return jnp.concatenate(rows, axis=0).astype(jnp.bfloat16)


def _tc_forward(emb2, weights, out_buf, blk0, n_total):
    n2 = emb2.shape[0]
    grid = n2 // (2 * BLK4)

    def body(xa_ref, xb_ref, wp_ref, wg0_ref, bg0_ref, wt0_ref, bt0_ref,
             wg1_ref, bg1_ref, wt1_ref, bt1_ref, *rest):
        o_ref = rest[-1]
        x4 = jnp.concatenate([xa_ref[...], xb_ref[...]], axis=1)
        h = jnp.dot(x4.astype(jnp.bfloat16), wp_ref[...],
                    preferred_element_type=jnp.float32)
        for wg, bg, wt, bt in ((wg0_ref, bg0_ref, wt0_ref, bt0_ref),
                               (wg1_ref, bg1_ref, wt1_ref, bt1_ref)):
            hb = h.astype(jnp.bfloat16)
            z = jnp.dot(hb, wg[...], preferred_element_type=jnp.float32) + bg[...]
            g = 1.0 / (1.0 + jnp.exp(-z))
            t = jnp.maximum(
                jnp.dot(hb, wt[...], preferred_element_type=jnp.float32)
                + bt[...], 0.0)
            h = g * t + (1.0 - g) * h
        o_ref[...] = jnp.concatenate(
            [h[:, c * DIM:(c + 1) * DIM] for c in range(4)], axis=0)

    w_spec = pl.BlockSpec((4 * DIM, 4 * DIM), lambda i: (0, 0))
    b_spec = pl.BlockSpec((1, 4 * DIM), lambda i: (0, 0))
    in_specs = [
        pl.BlockSpec((BLK4, 2 * DIM), lambda i: (2 * i, 0)),
        pl.BlockSpec((BLK4, 2 * DIM), lambda i: (2 * i + 1, 0)),
        w_spec, w_spec, b_spec, w_spec, b_spec,
        w_spec, b_spec, w_spec, b_spec,
    ]
    operands = [emb2, emb2, *weights]
    io_alias = {}
    if out_buf is not None:
        in_specs.append(pl.BlockSpec(memory_space=pl.ANY))
        operands.append(out_buf)
        io_alias = {len(operands) - 1: 0}

    return pl.pallas_call(
        body,
        grid=(grid,),
        in_specs=in_specs,
        out_specs=pl.BlockSpec((4 * BLK4, DIM), lambda i: (blk0 + i, 0)),
        out_shape=jax.ShapeDtypeStruct((n_total, DIM), jnp.float32),
        input_output_aliases=io_alias,
    )(*operands)


def kernel(x, word_vectors, W_proj, Wt0, bt0, Wg0, bg0, Wt1, bt1, Wg1, bg1):
    b, l = x.shape
    n = b * l
    per = n // Q
    nw = 32
    idx_flat = x.reshape(-1).astype(jnp.int32)
    idx4 = idx_flat.reshape(Q, per // CH, CH)

    def d4(v):
        return jnp.concatenate([v, v, v, v]).reshape(1, 4 * DIM)

    weights = (
        _pack4(W_proj.T),
        _pack4(Wg0.T), d4(bg0), _pack4(Wt0.T), d4(bt0),
        _pack4(Wg1.T), d4(bg1), _pack4(Wt1.T), d4(bt1),
    )

    embs = [_sc_gather(idx4[q], word_vectors) for q in range(Q)]
    out = None
    for q in range(Q):
        emb2 = embs[q].reshape(per // 2, 2 * DIM)
        out = _tc_forward(emb2, weights, out, q * (per // (4 * BLK4)), n)
    return out.reshape(b, l, DIM)
